# Initial kernel scaffold; baseline (speedup 1.0000x reference)
import functools

import jax
import jax.numpy as jnp
from jax import lax
from jax.experimental import pallas as pl
from jax.experimental.pallas import tpu as pltpu
from jax.experimental.pallas import tpu_sc as plsc

VOCAB = 1000000
EMBED = 50
CTX = 5
HIDDEN = 256
NUM_CLASSES = 64
BATCH = 16384

NC = 2
NS = 16
NW = NC * NS

N_IDX = BATCH * CTX
PER_W = N_IDX // NW
CHUNK = 128
N_CHUNKS = PER_W // CHUNK

_sc_mesh = plsc.VectorSubcoreMesh(core_axis_name="c", subcore_axis_name="s")


@functools.partial(
    pl.kernel,
    mesh=_sc_mesh,
    out_type=jax.ShapeDtypeStruct((N_IDX, EMBED), jnp.float32),
    scratch_types=[
        pltpu.VMEM((PER_W,), jnp.int32),
        pltpu.VMEM((2, CHUNK, EMBED), jnp.float32),
        pltpu.SemaphoreType.DMA,
        pltpu.SemaphoreType.DMA,
    ],
)
def _sc_gather(idx_hbm, table_hbm, out_hbm, idx_v, rows_v, gsem, osem):
    wid = lax.axis_index("s") * NC + lax.axis_index("c")
    base = wid * PER_W
    pltpu.sync_copy(idx_hbm.at[pl.ds(base, PER_W)], idx_v)
    gathers = []
    outs = []
    for j in range(N_CHUNKS):
        b = j % 2
        if j >= 2:
            outs[j - 2].wait()
        g = pltpu.async_copy(
            table_hbm.at[idx_v.at[pl.ds(j * CHUNK, CHUNK)]], rows_v.at[b], gsem)
        gathers.append(g)
        if j >= 1:
            gathers[j - 1].wait()
            o = pltpu.async_copy(
                rows_v.at[(j - 1) % 2],
                out_hbm.at[pl.ds(base + (j - 1) * CHUNK, CHUNK)], osem)
            outs.append(o)
    gathers[-1].wait()
    if N_CHUNKS >= 2:
        outs[-1].wait()
    pltpu.sync_copy(rows_v.at[(N_CHUNKS - 1) % 2],
                    out_hbm.at[pl.ds(base + (N_CHUNKS - 1) * CHUNK, CHUNK)])


_BB = 2048


def _mlp_body(emb_ref, w1_ref, b1_ref, w2_ref, b2_ref, out_ref):
    h = jnp.tanh(
        jnp.dot(emb_ref[...], w1_ref[...],
                preferred_element_type=jnp.float32,
                precision=lax.Precision.HIGHEST) + b1_ref[...])
    logits = jnp.dot(h, w2_ref[...],
                     preferred_element_type=jnp.float32,
                     precision=lax.Precision.HIGHEST) + b2_ref[...]
    m = jnp.max(logits, axis=1, keepdims=True)
    l = logits - m
    lse = jnp.log(jnp.sum(jnp.exp(l), axis=1, keepdims=True))
    out_ref[...] = l - lse


def _mlp(emb, W1, b1, W2, b2):
    grid = (BATCH // _BB,)
    return pl.pallas_call(
        _mlp_body,
        grid=grid,
        in_specs=[
            pl.BlockSpec((_BB, EMBED * CTX), lambda i: (i, 0)),
            pl.BlockSpec((EMBED * CTX, HIDDEN), lambda i: (0, 0)),
            pl.BlockSpec((1, HIDDEN), lambda i: (0, 0)),
            pl.BlockSpec((HIDDEN, NUM_CLASSES), lambda i: (0, 0)),
            pl.BlockSpec((1, NUM_CLASSES), lambda i: (0, 0)),
        ],
        out_specs=pl.BlockSpec((_BB, NUM_CLASSES), lambda i: (i, 0)),
        out_shape=jax.ShapeDtypeStruct((BATCH, NUM_CLASSES), jnp.float32),
    )(emb, W1, b1, W2, b2)


def kernel(x, table, W1, b1, W2, b2):
    flat_idx = x.reshape(-1).astype(jnp.int32)
    emb_flat = _sc_gather(flat_idx, table)
    emb = emb_flat.reshape(BATCH, EMBED * CTX)
    return _mlp(emb, W1, b1.reshape(1, HIDDEN), W2, b2.reshape(1, NUM_CLASSES))


# speedup vs baseline: 1.5228x; 1.5228x over previous
import functools

import jax
import jax.numpy as jnp
from jax import lax
from jax.experimental import pallas as pl
from jax.experimental.pallas import tpu as pltpu
from jax.experimental.pallas import tpu_sc as plsc

VOCAB = 1000000
EMBED = 50
CTX = 5
HIDDEN = 256
NUM_CLASSES = 64
BATCH = 16384

NC = 2
NS = 16
NW = NC * NS

N_IDX = BATCH * CTX
PER_W = N_IDX // NW
CHUNK = 128
N_CHUNKS = PER_W // CHUNK

EMBED_PAD = ((EMBED + 7) // 8) * 8
DRAIN = CHUNK * (EMBED_PAD - EMBED)

_sc_mesh = plsc.VectorSubcoreMesh(core_axis_name="c", subcore_axis_name="s")


@functools.partial(
    pl.kernel,
    mesh=_sc_mesh,
    out_type=jax.ShapeDtypeStruct((N_IDX, EMBED), jnp.float32),
    compiler_params=pltpu.CompilerParams(use_tc_tiling_on_sc=False),
    scratch_types=[
        pltpu.VMEM((N_CHUNKS, CHUNK), jnp.int32),
        pltpu.VMEM((CHUNK, EMBED), jnp.float32),
        pltpu.VMEM((DRAIN,), jnp.float32),
        pltpu.SemaphoreType.DMA,
    ],
)
def _sc_gather(idx_hbm, table_hbm, drain_hbm, out_hbm, idx_v, rows_v, drain_v, gsem):
    wid = lax.axis_index("s") * NC + lax.axis_index("c")
    base = wid * PER_W
    pltpu.sync_copy(idx_hbm.at[wid], idx_v)
    for j in range(N_CHUNKS):
        pltpu.async_copy(table_hbm.at[idx_v.at[j]], rows_v, gsem).wait()
        pltpu.sync_copy(rows_v, out_hbm.at[pl.ds(base + j * CHUNK, CHUNK)])


_BB = 2048


def _mlp_body(emb_ref, w1_ref, b1_ref, w2_ref, b2_ref, out_ref):
    h = jnp.tanh(
        jnp.dot(emb_ref[...], w1_ref[...],
                preferred_element_type=jnp.float32,
                precision=lax.Precision.HIGHEST) + b1_ref[...])
    logits = jnp.dot(h, w2_ref[...],
                     preferred_element_type=jnp.float32,
                     precision=lax.Precision.HIGHEST) + b2_ref[...]
    m = jnp.max(logits, axis=1, keepdims=True)
    l = logits - m
    lse = jnp.log(jnp.sum(jnp.exp(l), axis=1, keepdims=True))
    out_ref[...] = l - lse


def _mlp(emb, W1, b1, W2, b2):
    grid = (BATCH // _BB,)
    return pl.pallas_call(
        _mlp_body,
        grid=grid,
        in_specs=[
            pl.BlockSpec((_BB, EMBED * CTX), lambda i: (i, 0)),
            pl.BlockSpec((EMBED * CTX, HIDDEN), lambda i: (0, 0)),
            pl.BlockSpec((1, HIDDEN), lambda i: (0, 0)),
            pl.BlockSpec((HIDDEN, NUM_CLASSES), lambda i: (0, 0)),
            pl.BlockSpec((1, NUM_CLASSES), lambda i: (0, 0)),
        ],
        out_specs=pl.BlockSpec((_BB, NUM_CLASSES), lambda i: (i, 0)),
        out_shape=jax.ShapeDtypeStruct((BATCH, NUM_CLASSES), jnp.float32),
    )(emb, W1, b1, W2, b2)


def kernel(x, table, W1, b1, W2, b2):
    flat_idx = x.reshape(-1).astype(jnp.int32).reshape(NW, N_CHUNKS, CHUNK)
    drain = jnp.zeros((DRAIN,), jnp.float32)
    emb_flat = _sc_gather(flat_idx, table, drain)
    emb = emb_flat.reshape(BATCH, EMBED * CTX)
    return _mlp(emb, W1, b1.reshape(1, HIDDEN), W2, b2.reshape(1, NUM_CLASSES))
